# Initial kernel scaffold; baseline (speedup 1.0000x reference)
import functools

import jax
import jax.numpy as jnp
from jax import lax
from jax.experimental import pallas as pl
from jax.experimental.pallas import tpu as pltpu
from jax.experimental.pallas import tpu_sc as plsc

N = 10000
E = 320000
IN = 128
OUT = 128
H = 4

NC = 2
NS = 16
NW = NC * NS
EPW = E // NW
NPW = N // NS
C = 80
LEAK = 0.2


def _proj_body(x_ref, wt_ref, wlr_ref, h_ref, s_ref):
    xb = x_ref[...]
    h_ref[...] = lax.dot_general(
        xb, wt_ref[...], (((1,), (0,)), ((), ())),
        preferred_element_type=jnp.float32)
    s_ref[...] = lax.dot_general(
        xb, wlr_ref[...], (((1,), (0,)), ((), ())),
        preferred_element_type=jnp.float32)


def _proj(x, wt, wlr):
    B = 2000
    return pl.pallas_call(
        _proj_body,
        grid=(N // B,),
        in_specs=[
            pl.BlockSpec((B, IN), lambda i: (i, 0)),
            pl.BlockSpec((IN, H * OUT), lambda i: (0, 0)),
            pl.BlockSpec((IN, 2 * H), lambda i: (0, 0)),
        ],
        out_specs=[
            pl.BlockSpec((B, H * OUT), lambda i: (i, 0)),
            pl.BlockSpec((B, 2 * H), lambda i: (i, 0)),
        ],
        out_shape=[
            jax.ShapeDtypeStruct((N, H * OUT), jnp.float32),
            jax.ShapeDtypeStruct((N, 2 * H), jnp.float32),
        ],
    )(x, wt, wlr)


def _recip_body(s_ref, r_ref):
    r_ref[...] = 1.0 / (s_ref[0] + s_ref[1] + 1e-16)


def _recip(sacc):
    B = 2000
    return pl.pallas_call(
        _recip_body,
        grid=(N // B,),
        in_specs=[pl.BlockSpec((NC, B, H), lambda i: (0, i, 0))],
        out_specs=pl.BlockSpec((B, H), lambda i: (i, 0)),
        out_shape=jax.ShapeDtypeStruct((N, H), jnp.float32),
    )(sacc)


def _merge_body(p_ref, o_ref):
    o_ref[...] = (p_ref[0] + p_ref[1]) * (1.0 / H)


def _merge(outp):
    B = 2000
    return pl.pallas_call(
        _merge_body,
        grid=(N // B,),
        in_specs=[pl.BlockSpec((NC, B, OUT), lambda i: (0, i, 0))],
        out_specs=pl.BlockSpec((B, OUT), lambda i: (i, 0)),
        out_shape=jax.ShapeDtypeStruct((N, OUT), jnp.float32),
    )(outp)


def _phase1(scores, row, col, z4):
    mesh = plsc.VectorSubcoreMesh(core_axis_name="c", subcore_axis_name="s")

    @functools.partial(
        pl.kernel,
        out_type=[
            jax.ShapeDtypeStruct((E, H), jnp.float32),
            jax.ShapeDtypeStruct((NC, N, H), jnp.float32),
        ],
        mesh=mesh,
        scratch_types=[
            pltpu.VMEM((N, 2 * H), jnp.float32),
            pltpu.VMEM((C,), jnp.int32),
            pltpu.VMEM((C,), jnp.int32),
            pltpu.VMEM((C, H), jnp.float32),
            pltpu.VMEM_SHARED((N, H), jnp.float32),
        ],
    )
    def k(scores_hbm, row_hbm, col_hbm, z4_hbm, enum_hbm, sacc_hbm,
          tbl, rowv, colv, ev, sacc_sh):
        cid = lax.axis_index("c")
        sid = lax.axis_index("s")
        wid = sid * NC + cid
        pltpu.sync_copy(scores_hbm, tbl)
        pltpu.sync_copy(z4_hbm.at[pl.ds(sid * NPW, NPW)],
                        sacc_sh.at[pl.ds(sid * NPW, NPW)])
        plsc.subcore_barrier()

        base = wid * EPW
        lane = lax.iota(jnp.int32, 16)

        @pl.loop(0, EPW, step=C)
        def _chunk(off):
            pltpu.sync_copy(row_hbm.at[pl.ds(base + off, C)], rowv)
            pltpu.sync_copy(col_hbm.at[pl.ds(base + off, C)], colv)

            @pl.loop(0, C, step=16)
            def _grp(i):
                r16 = rowv[pl.ds(i, 16)]
                c16 = colv[pl.ds(i, 16)]
                for hh in range(H):
                    hv = jnp.full((16,), hh, jnp.int32)
                    sl = plsc.load_gather(tbl, [r16, hv])
                    sr = plsc.load_gather(tbl, [c16, hv + H])
                    l = sl + sr
                    l = jnp.where(l >= 0.0, l, l * LEAK)
                    plsc.store_scatter(ev, [lane + i, hv], jnp.exp(l))

            pltpu.sync_copy(ev, enum_hbm.at[pl.ds(base + off, C)])
            pltpu.sync_copy(ev, sacc_sh.at[rowv], add=True)

        plsc.subcore_barrier()
        pltpu.sync_copy(sacc_sh.at[pl.ds(sid * NPW, NPW)],
                        sacc_hbm.at[cid].at[pl.ds(sid * NPW, NPW)])

    return k(scores, row, col, z4)


def _phase2(h, rtab, enum, row, col, z128):
    mesh = plsc.VectorSubcoreMesh(core_axis_name="c", subcore_axis_name="s")

    @functools.partial(
        pl.kernel,
        out_type=[
            jax.ShapeDtypeStruct((E, H), jnp.float32),
            jax.ShapeDtypeStruct((NC, N, OUT), jnp.float32),
        ],
        mesh=mesh,
        scratch_types=[
            pltpu.VMEM((N, H), jnp.float32),
            pltpu.VMEM((C,), jnp.int32),
            pltpu.VMEM((C,), jnp.int32),
            pltpu.VMEM((C, H), jnp.float32),
            pltpu.VMEM((C, H), jnp.float32),
            pltpu.VMEM((C, H * OUT), jnp.float32),
            pltpu.VMEM((C, OUT), jnp.float32),
            pltpu.VMEM_SHARED((N, OUT), jnp.float32),
        ],
    )
    def k(h_hbm, rtab_hbm, enum_hbm, row_hbm, col_hbm, z128_hbm,
          alpha_hbm, outp_hbm,
          rtab, rowv, colv, ev, av, rows, contrib, acc_sh):
        cid = lax.axis_index("c")
        sid = lax.axis_index("s")
        wid = sid * NC + cid
        pltpu.sync_copy(rtab_hbm, rtab)
        pltpu.sync_copy(z128_hbm.at[pl.ds(sid * NPW, NPW)],
                        acc_sh.at[pl.ds(sid * NPW, NPW)])
        plsc.subcore_barrier()

        base = wid * EPW
        lane = lax.iota(jnp.int32, 16)

        @pl.loop(0, EPW, step=C)
        def _chunk(off):
            pltpu.sync_copy(row_hbm.at[pl.ds(base + off, C)], rowv)
            pltpu.sync_copy(col_hbm.at[pl.ds(base + off, C)], colv)
            pltpu.sync_copy(enum_hbm.at[pl.ds(base + off, C)], ev)
            pltpu.sync_copy(h_hbm.at[colv], rows)

            @pl.loop(0, C, step=16)
            def _grp(i):
                r16 = rowv[pl.ds(i, 16)]
                for hh in range(H):
                    hv = jnp.full((16,), hh, jnp.int32)
                    rv = plsc.load_gather(rtab, [r16, hv])
                    en = plsc.load_gather(ev, [lane + i, hv])
                    plsc.store_scatter(av, [lane + i, hv], en * rv)

            pltpu.sync_copy(av, alpha_hbm.at[pl.ds(base + off, C)])

            @pl.loop(0, C)
            def _edge(e):
                ev16 = jnp.full((16,), e, jnp.int32)
                ab = [plsc.load_gather(av, [ev16, jnp.full((16,), hh, jnp.int32)])
                      for hh in range(H)]
                for j in range(OUT // 16):
                    acc = ab[0] * rows[e, pl.ds(j * 16, 16)]
                    for hh in range(1, H):
                        acc = acc + ab[hh] * rows[e, pl.ds(hh * OUT + j * 16, 16)]
                    contrib[e, pl.ds(j * 16, 16)] = acc

            pltpu.sync_copy(contrib, acc_sh.at[rowv], add=True)

        plsc.subcore_barrier()
        pltpu.sync_copy(acc_sh.at[pl.ds(sid * NPW, NPW)],
                        outp_hbm.at[cid].at[pl.ds(sid * NPW, NPW)])

    return k(h, rtab, enum, row, col, z128)


def kernel(x, edge_index, W, attn):
    row = edge_index[0]
    col = edge_index[1]
    wt = W.T
    attn_l = attn[:, :OUT]
    attn_r = attn[:, OUT:]
    w3 = wt.reshape(IN, H, OUT)
    wl = jnp.einsum("khj,hj->kh", w3, attn_l)
    wr = jnp.einsum("khj,hj->kh", w3, attn_r)
    wlr = jnp.concatenate([wl, wr], axis=1)

    h, scores = _proj(x, wt, wlr)
    z4 = jnp.zeros((N, H), jnp.float32)
    z128 = jnp.zeros((N, OUT), jnp.float32)

    enum, sacc = _phase1(scores, row, col, z4)
    rtab = _recip(sacc)
    alpha, outp = _phase2(h, rtab, enum, row, col, z128)
    out = _merge(outp)
    return out, alpha


# speedup vs baseline: 19.1845x; 19.1845x over previous
import functools

import jax
import jax.numpy as jnp
from jax import lax
from jax.experimental import pallas as pl
from jax.experimental.pallas import tpu as pltpu
from jax.experimental.pallas import tpu_sc as plsc

N = 10000
E = 320000
IN = 128
OUT = 128
H = 4

NC = 2
NS = 16
NW = NC * NS
EPW = E // NW
NPW = N // NS
C = 80
LEAK = 0.2


def _proj_body(x_ref, wt_ref, wlr_ref, h_ref, s_ref):
    xb = x_ref[...]
    h_ref[...] = lax.dot_general(
        xb, wt_ref[...], (((1,), (0,)), ((), ())),
        preferred_element_type=jnp.float32)
    s_ref[...] = lax.dot_general(
        xb, wlr_ref[...], (((1,), (0,)), ((), ())),
        preferred_element_type=jnp.float32)


def _proj(x, wt, wlr):
    B = 2000
    return pl.pallas_call(
        _proj_body,
        grid=(N // B,),
        in_specs=[
            pl.BlockSpec((B, IN), lambda i: (i, 0)),
            pl.BlockSpec((IN, H * OUT), lambda i: (0, 0)),
            pl.BlockSpec((IN, 2 * H), lambda i: (0, 0)),
        ],
        out_specs=[
            pl.BlockSpec((B, H * OUT), lambda i: (i, 0)),
            pl.BlockSpec((B, 2 * H), lambda i: (i, 0)),
        ],
        out_shape=[
            jax.ShapeDtypeStruct((N, H * OUT), jnp.float32),
            jax.ShapeDtypeStruct((N, 2 * H), jnp.float32),
        ],
    )(x, wt, wlr)


def _recip_body(s_ref, r_ref):
    r_ref[...] = 1.0 / (s_ref[0] + s_ref[1] + 1e-16)


def _recip(sacc):
    B = 2000
    return pl.pallas_call(
        _recip_body,
        grid=(N // B,),
        in_specs=[pl.BlockSpec((NC, B, 4 * H), lambda i: (0, i, 0))],
        out_specs=pl.BlockSpec((B, 4 * H), lambda i: (i, 0)),
        out_shape=jax.ShapeDtypeStruct((N, 4 * H), jnp.float32),
    )(sacc)


def _merge_body(p_ref, o_ref):
    o_ref[...] = (p_ref[0] + p_ref[1]) * (1.0 / H)


def _merge(outp):
    B = 2000
    return pl.pallas_call(
        _merge_body,
        grid=(N // B,),
        in_specs=[pl.BlockSpec((NC, B, OUT), lambda i: (0, i, 0))],
        out_specs=pl.BlockSpec((B, OUT), lambda i: (i, 0)),
        out_shape=jax.ShapeDtypeStruct((N, OUT), jnp.float32),
    )(outp)


def _phase1(scores, row, col, z16):
    mesh = plsc.VectorSubcoreMesh(core_axis_name="c", subcore_axis_name="s", num_cores=NC, num_subcores=NS)

    @functools.partial(
        pl.kernel,
        out_type=[
            jax.ShapeDtypeStruct((E, H), jnp.float32),
            jax.ShapeDtypeStruct((NC, N, 4 * H), jnp.float32),
        ],
        mesh=mesh,
        compiler_params=pltpu.CompilerParams(use_tc_tiling_on_sc=False, needs_layout_passes=False),
        scratch_types=[
            pltpu.VMEM((N, 2 * H), jnp.float32),
            pltpu.VMEM((C,), jnp.int32),
            pltpu.VMEM((C,), jnp.int32),
            pltpu.VMEM((C, H), jnp.float32),
            pltpu.VMEM((C, 4 * H), jnp.float32),
            pltpu.VMEM_SHARED((N, 4 * H), jnp.float32),
        ],
    )
    def k(scores_hbm, row_hbm, col_hbm, z16_hbm, enum_hbm, sacc_hbm,
          tbl, rowv, colv, ev, evp, sacc_sh):
        cid = lax.axis_index("c")
        sid = lax.axis_index("s")
        wid = sid * NC + cid
        pltpu.sync_copy(scores_hbm, tbl)
        pltpu.sync_copy(z16_hbm.at[pl.ds(sid * NPW, NPW)],
                        sacc_sh.at[pl.ds(sid * NPW, NPW)])
        @pl.loop(0, C)
        def _z(i):
            evp[i, pl.ds(0, 16)] = jnp.zeros((16,), jnp.float32)

        plsc.subcore_barrier()

        base = wid * EPW
        lane = lax.iota(jnp.int32, 16)

        @pl.loop(0, EPW, step=C)
        def _chunk(off):
            pltpu.sync_copy(row_hbm.at[pl.ds(base + off, C)], rowv)
            pltpu.sync_copy(col_hbm.at[pl.ds(base + off, C)], colv)

            @pl.loop(0, C, step=16)
            def _grp(i):
                r16 = rowv[pl.ds(i, 16)]
                c16 = colv[pl.ds(i, 16)]
                for hh in range(H):
                    hv = jnp.full((16,), hh, jnp.int32)
                    sl = plsc.load_gather(tbl, [r16, hv])
                    sr = plsc.load_gather(tbl, [c16, hv + H])
                    l = sl + sr
                    l = jnp.where(l >= 0.0, l, l * LEAK)
                    e = jnp.exp(l)
                    plsc.store_scatter(ev, [lane + i, hv], e)
                    plsc.store_scatter(evp, [lane + i, hv], e)

            pltpu.sync_copy(ev, enum_hbm.at[pl.ds(base + off, C)])
            pltpu.sync_copy(evp, sacc_sh.at[rowv], add=True)

        plsc.subcore_barrier()
        pltpu.sync_copy(sacc_sh.at[pl.ds(sid * NPW, NPW)],
                        sacc_hbm.at[cid].at[pl.ds(sid * NPW, NPW)])

    return k(scores, row, col, z16)


def _phase2(h, rtab, enum, row, col, z128):
    mesh = plsc.VectorSubcoreMesh(core_axis_name="c", subcore_axis_name="s", num_cores=NC, num_subcores=NS)

    @functools.partial(
        pl.kernel,
        out_type=[
            jax.ShapeDtypeStruct((E, H), jnp.float32),
            jax.ShapeDtypeStruct((NC, N, OUT), jnp.float32),
        ],
        mesh=mesh,
        compiler_params=pltpu.CompilerParams(use_tc_tiling_on_sc=False, needs_layout_passes=False),
        scratch_types=[
            pltpu.VMEM((C,), jnp.int32),
            pltpu.VMEM((2, C // 2), jnp.int32),
            pltpu.VMEM((2, C // 2), jnp.int32),
            pltpu.VMEM((C, H), jnp.float32),
            pltpu.VMEM((C, H), jnp.float32),
            pltpu.VMEM((C, 4 * H), jnp.float32),
            pltpu.VMEM((C // 2, H * OUT), jnp.float32),
            pltpu.VMEM((C // 2, OUT), jnp.float32),
            pltpu.VMEM_SHARED((N, OUT), jnp.float32),
        ],
    )
    def k(h_hbm, rtab_hbm, enum_hbm, row_hbm, col_hbm, z128_hbm,
          alpha_hbm, outp_hbm,
          rowv, rowv2, colv2, ev, av, rgath, rows, contrib, acc_sh):
        cid = lax.axis_index("c")
        sid = lax.axis_index("s")
        wid = sid * NC + cid
        pltpu.sync_copy(z128_hbm.at[pl.ds(sid * NPW, NPW)],
                        acc_sh.at[pl.ds(sid * NPW, NPW)])
        plsc.subcore_barrier()

        base = wid * EPW
        lane = lax.iota(jnp.int32, 16)

        @pl.loop(0, EPW, step=C)
        def _chunk(off):
            pltpu.sync_copy(row_hbm.at[pl.ds(base + off, C)], rowv)
            pltpu.sync_copy(row_hbm.at[pl.ds(base + off, C // 2)], rowv2.at[0])
            pltpu.sync_copy(row_hbm.at[pl.ds(base + off + C // 2, C // 2)],
                            rowv2.at[1])
            pltpu.sync_copy(col_hbm.at[pl.ds(base + off, C // 2)], colv2.at[0])
            pltpu.sync_copy(col_hbm.at[pl.ds(base + off + C // 2, C // 2)],
                            colv2.at[1])
            pltpu.sync_copy(enum_hbm.at[pl.ds(base + off, C)], ev)
            pltpu.sync_copy(rtab_hbm.at[rowv], rgath)

            @pl.loop(0, C, step=16)
            def _grp(i):
                for hh in range(H):
                    hv = jnp.full((16,), hh, jnp.int32)
                    rv = plsc.load_gather(rgath, [lane + i, hv])
                    en = plsc.load_gather(ev, [lane + i, hv])
                    plsc.store_scatter(av, [lane + i, hv], en * rv)

            pltpu.sync_copy(av, alpha_hbm.at[pl.ds(base + off, C)])

            for half in range(2):
                pltpu.sync_copy(h_hbm.at[colv2.at[half]], rows)

                @pl.loop(0, C // 2)
                def _edge(e, _h=half):
                    ge = _h * (C // 2) + e
                    ev16 = jnp.full((16,), ge, jnp.int32)
                    ab = [plsc.load_gather(
                              av, [ev16, jnp.full((16,), hh, jnp.int32)])
                          for hh in range(H)]
                    for j in range(OUT // 16):
                        acc = ab[0] * rows[e, pl.ds(j * 16, 16)]
                        for hh in range(1, H):
                            acc = acc + ab[hh] * rows[e, pl.ds(hh * OUT + j * 16, 16)]
                        contrib[e, pl.ds(j * 16, 16)] = acc

                pltpu.sync_copy(contrib, acc_sh.at[rowv2.at[half]], add=True)

        plsc.subcore_barrier()
        pltpu.sync_copy(acc_sh.at[pl.ds(sid * NPW, NPW)],
                        outp_hbm.at[cid].at[pl.ds(sid * NPW, NPW)])

    return k(h, rtab, enum, row, col, z128)


def kernel(x, edge_index, W, attn):
    row = edge_index[0]
    col = edge_index[1]
    wt = W.T
    attn_l = attn[:, :OUT]
    attn_r = attn[:, OUT:]
    w3 = wt.reshape(IN, H, OUT)
    wl = jnp.einsum("khj,hj->kh", w3, attn_l)
    wr = jnp.einsum("khj,hj->kh", w3, attn_r)
    wlr = jnp.concatenate([wl, wr], axis=1)

    h, scores = _proj(x, wt, wlr)
    z16 = jnp.zeros((N, 4 * H), jnp.float32)
    z128 = jnp.zeros((N, OUT), jnp.float32)

    enum, sacc = _phase1(scores, row, col, z16)
    rtab = _recip(sacc)
    alpha, outp = _phase2(h, rtab, enum, row, col, z128)
    out = _merge(outp)
    return out, alpha


# speedup vs baseline: 31.0346x; 1.6177x over previous
import functools

import jax
import jax.numpy as jnp
from jax import lax
from jax.experimental import pallas as pl
from jax.experimental.pallas import tpu as pltpu
from jax.experimental.pallas import tpu_sc as plsc

N = 10000
E = 320000
IN = 128
OUT = 128
H = 4

NC = 2
NS = 16
NW = NC * NS
EPW = E // NW
NPW = N // NS
C = 80
LEAK = 0.2


def _proj_body(x_ref, wt_ref, wlr_ref, h_ref, s_ref):
    xb = x_ref[...]
    h_ref[...] = lax.dot_general(
        xb, wt_ref[...], (((1,), (0,)), ((), ())),
        preferred_element_type=jnp.float32)
    s_ref[...] = lax.dot_general(
        xb, wlr_ref[...], (((1,), (0,)), ((), ())),
        preferred_element_type=jnp.float32)


def _proj(x, wt, wlr):
    B = 2000
    return pl.pallas_call(
        _proj_body,
        grid=(N // B,),
        in_specs=[
            pl.BlockSpec((B, IN), lambda i: (i, 0)),
            pl.BlockSpec((IN, H * OUT), lambda i: (0, 0)),
            pl.BlockSpec((IN, 2 * H), lambda i: (0, 0)),
        ],
        out_specs=[
            pl.BlockSpec((B, H * OUT), lambda i: (i, 0)),
            pl.BlockSpec((B, 2 * H), lambda i: (i, 0)),
        ],
        out_shape=[
            jax.ShapeDtypeStruct((N, H * OUT), jnp.float32),
            jax.ShapeDtypeStruct((N, 2 * H), jnp.float32),
        ],
    )(x, wt, wlr)


def _recip_body(s_ref, r_ref):
    r_ref[...] = 1.0 / (s_ref[0] + s_ref[1] + 1e-16)


def _recip(sacc):
    B = 2000
    return pl.pallas_call(
        _recip_body,
        grid=(N // B,),
        in_specs=[pl.BlockSpec((NC, B, 4 * H), lambda i: (0, i, 0))],
        out_specs=pl.BlockSpec((B, 4 * H), lambda i: (i, 0)),
        out_shape=jax.ShapeDtypeStruct((N, 4 * H), jnp.float32),
    )(sacc)


def _merge_body(p_ref, o_ref):
    o_ref[...] = (p_ref[0] + p_ref[1]) * (1.0 / H)


def _merge(outp):
    B = 2000
    return pl.pallas_call(
        _merge_body,
        grid=(N // B,),
        in_specs=[pl.BlockSpec((NC, B, OUT), lambda i: (0, i, 0))],
        out_specs=pl.BlockSpec((B, OUT), lambda i: (i, 0)),
        out_shape=jax.ShapeDtypeStruct((N, OUT), jnp.float32),
    )(outp)


def _phase1(scores, epk1, z16):
    mesh = plsc.VectorSubcoreMesh(core_axis_name="c", subcore_axis_name="s", num_cores=NC, num_subcores=NS)
    NCH = EPW // C

    @functools.partial(
        pl.kernel,
        out_type=[
            jax.ShapeDtypeStruct((E, H), jnp.float32),
            jax.ShapeDtypeStruct((NC, N, 4 * H), jnp.float32),
        ],
        mesh=mesh,
        compiler_params=pltpu.CompilerParams(use_tc_tiling_on_sc=False, needs_layout_passes=False),
        scratch_types=[
            pltpu.VMEM((N, 2 * H), jnp.float32),
            pltpu.VMEM((2, 2, C), jnp.int32),
            pltpu.VMEM((C, H), jnp.float32),
            pltpu.VMEM((C, 4 * H), jnp.float32),
            pltpu.VMEM_SHARED((N, 4 * H), jnp.float32),
            pltpu.SemaphoreType.DMA,
            pltpu.SemaphoreType.DMA,
        ],
    )
    def k(scores_hbm, epk_hbm, z16_hbm, enum_hbm, sacc_hbm,
          tbl, eb, ev, evp, sacc_sh, se0, se1):
        cid = lax.axis_index("c")
        sid = lax.axis_index("s")
        wid = sid * NC + cid
        pltpu.sync_copy(scores_hbm, tbl)
        pltpu.sync_copy(z16_hbm.at[pl.ds(sid * NPW, NPW)],
                        sacc_sh.at[pl.ds(sid * NPW, NPW)])
        @pl.loop(0, C)
        def _z(i):
            evp[i, pl.ds(0, 16)] = jnp.zeros((16,), jnp.float32)

        plsc.subcore_barrier()

        base = wid * EPW
        lane = lax.iota(jnp.int32, 16)
        my_epk = epk_hbm.at[wid]
        sems_e = (se0, se1)

        def do_chunk(ck, cur, nxt, prefetch):
            if prefetch:
                pltpu.async_copy(my_epk.at[ck + 1], eb.at[nxt], sems_e[nxt])

            @pl.loop(0, C, step=16)
            def _grp(i):
                r16 = eb[cur, 0, pl.ds(i, 16)]
                c16 = eb[cur, 1, pl.ds(i, 16)]
                for hh in range(H):
                    hv = jnp.full((16,), hh, jnp.int32)
                    sl = plsc.load_gather(tbl, [r16, hv])
                    sr = plsc.load_gather(tbl, [c16, hv + H])
                    l = sl + sr
                    l = jnp.where(l >= 0.0, l, l * LEAK)
                    e = jnp.exp(l)
                    plsc.store_scatter(ev, [lane + i, hv], e)
                    plsc.store_scatter(evp, [lane + i, hv], e)

            pltpu.sync_copy(ev, enum_hbm.at[pl.ds(base + ck * C, C)])
            pltpu.sync_copy(evp, sacc_sh.at[eb.at[cur].at[0]], add=True)
            if prefetch:
                pltpu.make_async_copy(my_epk.at[0], eb.at[nxt],
                                      sems_e[nxt]).wait()

        pltpu.sync_copy(my_epk.at[0], eb.at[0])

        @pl.loop(0, NCH - 1, step=2)
        def _pair(g):
            do_chunk(g, 0, 1, True)
            do_chunk(g + 1, 1, 0, True)

        do_chunk(NCH - 1, 0, 1, False)

        plsc.subcore_barrier()
        pltpu.sync_copy(sacc_sh.at[pl.ds(sid * NPW, NPW)],
                        sacc_hbm.at[cid].at[pl.ds(sid * NPW, NPW)])

    return k(scores, epk1, z16)


def _phase2(h, rtab, enum, epk2, z128):
    mesh = plsc.VectorSubcoreMesh(core_axis_name="c", subcore_axis_name="s", num_cores=NC, num_subcores=NS)
    HC = C // 2
    NCH = EPW // C

    @functools.partial(
        pl.kernel,
        out_type=[
            jax.ShapeDtypeStruct((E, H), jnp.float32),
            jax.ShapeDtypeStruct((NC, N, OUT), jnp.float32),
        ],
        mesh=mesh,
        compiler_params=pltpu.CompilerParams(use_tc_tiling_on_sc=False, needs_layout_passes=False),
        scratch_types=[
            pltpu.VMEM((2, 4, HC), jnp.int32),
            pltpu.VMEM((2, C, H), jnp.float32),
            pltpu.VMEM((2, C, 4 * H), jnp.float32),
            pltpu.VMEM((HC, H * OUT), jnp.float32),
            pltpu.VMEM((HC, H * OUT), jnp.float32),
            pltpu.VMEM((HC, OUT), jnp.float32),
            pltpu.VMEM_SHARED((N, OUT), jnp.float32),
            pltpu.SemaphoreType.DMA,
            pltpu.SemaphoreType.DMA,
            pltpu.SemaphoreType.DMA,
            pltpu.SemaphoreType.DMA,
            pltpu.SemaphoreType.DMA,
            pltpu.SemaphoreType.DMA,
        ],
    )
    def k(h_hbm, rtab_hbm, enum_hbm, epk_hbm, z128_hbm,
          alpha_hbm, outp_hbm,
          ebuf, evav, rgath, rows0, rows1, contrib, acc_sh,
          se0, se1, sp0, sp1, sr0, sr1):
        cid = lax.axis_index("c")
        sid = lax.axis_index("s")
        wid = sid * NC + cid
        sems_e = (se0, se1)
        sems_p = (sp0, sp1)
        sems_r = (sr0, sr1)
        rowsb = (rows0, rows1)
        pltpu.sync_copy(z128_hbm.at[pl.ds(sid * NPW, NPW)],
                        acc_sh.at[pl.ds(sid * NPW, NPW)])
        plsc.subcore_barrier()

        base = wid * EPW
        lane = lax.iota(jnp.int32, 16)
        my_epk = epk_hbm.at[wid]

        def issue_pre(ck, s):
            pltpu.async_copy(enum_hbm.at[pl.ds(base + ck * C, C)],
                             evav.at[s], sems_p[s])
            pltpu.async_copy(rtab_hbm.at[ebuf.at[s].at[0]],
                             rgath.at[s].at[pl.ds(0, HC)], sems_p[s])
            pltpu.async_copy(rtab_hbm.at[ebuf.at[s].at[1]],
                             rgath.at[s].at[pl.ds(HC, HC)], sems_p[s])

        def wait_pre(s):
            pltpu.make_async_copy(enum_hbm.at[pl.ds(0, C)],
                                  evav.at[s], sems_p[s]).wait()
            pltpu.make_async_copy(rtab_hbm.at[ebuf.at[s].at[0]],
                                  rgath.at[s].at[pl.ds(0, HC)], sems_p[s]).wait()
            pltpu.make_async_copy(rtab_hbm.at[ebuf.at[s].at[1]],
                                  rgath.at[s].at[pl.ds(HC, HC)], sems_p[s]).wait()

        def issue_rows(s, hf):
            pltpu.async_copy(h_hbm.at[ebuf.at[s].at[2 + hf]],
                             rowsb[hf], sems_r[hf])

        def wait_rows(s, hf):
            pltpu.make_async_copy(h_hbm.at[ebuf.at[s].at[2 + hf]],
                                  rowsb[hf], sems_r[hf]).wait()

        def issue_ebuf(ck, s):
            pltpu.async_copy(my_epk.at[ck], ebuf.at[s], sems_e[s])

        def wait_ebuf(s):
            pltpu.make_async_copy(my_epk.at[0], ebuf.at[s], sems_e[s]).wait()

        def do_chunk(ck, cur, nxt, prefetch, prefetch2):
            if prefetch:
                wait_ebuf(nxt)
                issue_pre(ck + 1, nxt)
            wait_pre(cur)

            @pl.loop(0, C, step=16)
            def _grp(i):
                for hh in range(H):
                    hv = jnp.full((16,), hh, jnp.int32)
                    rv = plsc.load_gather(rgath.at[cur], [lane + i, hv])
                    en = plsc.load_gather(evav.at[cur], [lane + i, hv])
                    plsc.store_scatter(evav.at[cur], [lane + i, hv], en * rv)

            pltpu.sync_copy(evav.at[cur], alpha_hbm.at[pl.ds(base + ck * C, C)])

            for hf in range(2):
                wait_rows(cur, hf)

                @pl.loop(0, HC)
                def _edge(e, _h=hf):
                    ge = _h * HC + e
                    ev16 = jnp.full((16,), ge, jnp.int32)
                    ab = [plsc.load_gather(
                              evav.at[cur],
                              [ev16, jnp.full((16,), hh, jnp.int32)])
                          for hh in range(H)]
                    rr = rowsb[_h]
                    for j in range(OUT // 16):
                        acc = ab[0] * rr[e, pl.ds(j * 16, 16)]
                        for hh in range(1, H):
                            acc = acc + ab[hh] * rr[e, pl.ds(hh * OUT + j * 16, 16)]
                        contrib[e, pl.ds(j * 16, 16)] = acc

                pltpu.sync_copy(contrib, acc_sh.at[ebuf.at[cur].at[hf]],
                                add=True)
                if prefetch:
                    issue_rows(nxt, hf)
            if prefetch2 is not None:
                issue_ebuf(prefetch2, cur)

        pltpu.sync_copy(my_epk.at[0], ebuf.at[0])
        issue_pre(0, 0)
        issue_rows(0, 0)
        issue_rows(0, 1)
        issue_ebuf(1, 1)

        @pl.loop(0, NCH - 1, step=2)
        def _pair(g):
            do_chunk(g, 0, 1, True, g + 2)
            do_chunk(g + 1, 1, 0, True, jnp.minimum(g + 3, NCH - 1))

        do_chunk(NCH - 1, 0, 1, False, None)
        wait_ebuf(1)

        plsc.subcore_barrier()
        pltpu.sync_copy(acc_sh.at[pl.ds(sid * NPW, NPW)],
                        outp_hbm.at[cid].at[pl.ds(sid * NPW, NPW)])

    return k(h, rtab, enum, epk2, z128)


def kernel(x, edge_index, W, attn):
    row = edge_index[0]
    col = edge_index[1]
    wt = W.T
    attn_l = attn[:, :OUT]
    attn_r = attn[:, OUT:]
    w3 = wt.reshape(IN, H, OUT)
    wl = jnp.einsum("khj,hj->kh", w3, attn_l)
    wr = jnp.einsum("khj,hj->kh", w3, attn_r)
    wlr = jnp.concatenate([wl, wr], axis=1)

    NCH = EPW // C
    HC = C // 2
    r4 = row.reshape(NW, NCH, 2, HC)
    c4 = col.reshape(NW, NCH, 2, HC)
    epk1 = jnp.concatenate([row.reshape(NW, NCH, 1, C),
                            col.reshape(NW, NCH, 1, C)], axis=2)
    epk2 = jnp.concatenate([r4, c4], axis=2)

    h, scores = _proj(x, wt, wlr)
    z16 = jnp.zeros((N, 4 * H), jnp.float32)
    z128 = jnp.zeros((N, OUT), jnp.float32)

    enum, sacc = _phase1(scores, epk1, z16)
    rtab = _recip(sacc)
    alpha, outp = _phase2(h, rtab, enum, epk2, z128)
    out = _merge(outp)
    return out, alpha


# speedup vs baseline: 38.0323x; 1.2255x over previous
import functools

import jax
import jax.numpy as jnp
from jax import lax
from jax.experimental import pallas as pl
from jax.experimental.pallas import tpu as pltpu
from jax.experimental.pallas import tpu_sc as plsc

N = 10000
E = 320000
IN = 128
OUT = 128
H = 4

NC = 2
NS = 16
NW = NC * NS
EPW = E // NW
NPW = N // NS
C = 80
LEAK = 0.2


def _proj_body(x_ref, wt_ref, wlr_ref, h_ref, s_ref):
    xb = x_ref[...]
    h_ref[...] = lax.dot_general(
        xb, wt_ref[...], (((1,), (0,)), ((), ())),
        preferred_element_type=jnp.float32)
    s_ref[...] = lax.dot_general(
        xb, wlr_ref[...], (((1,), (0,)), ((), ())),
        preferred_element_type=jnp.float32)


def _proj(x, wt, wlr):
    B = 2000
    return pl.pallas_call(
        _proj_body,
        grid=(N // B,),
        in_specs=[
            pl.BlockSpec((B, IN), lambda i: (i, 0)),
            pl.BlockSpec((IN, H * OUT), lambda i: (0, 0)),
            pl.BlockSpec((IN, 2 * H), lambda i: (0, 0)),
        ],
        out_specs=[
            pl.BlockSpec((B, H * OUT), lambda i: (i, 0)),
            pl.BlockSpec((B, 2 * H), lambda i: (i, 0)),
        ],
        out_shape=[
            jax.ShapeDtypeStruct((N, H * OUT), jnp.float32),
            jax.ShapeDtypeStruct((N, 2 * H), jnp.float32),
        ],
    )(x, wt, wlr)


def _recip_body(s_ref, r_ref):
    r_ref[...] = 1.0 / (s_ref[0] + s_ref[1] + 1e-16)


def _recip(sacc):
    B = 2000
    return pl.pallas_call(
        _recip_body,
        grid=(N // B,),
        in_specs=[pl.BlockSpec((NC, B, 4 * H), lambda i: (0, i, 0))],
        out_specs=pl.BlockSpec((B, 4 * H), lambda i: (i, 0)),
        out_shape=jax.ShapeDtypeStruct((N, 4 * H), jnp.float32),
    )(sacc)


def _merge_body(p_ref, o_ref):
    o_ref[...] = (p_ref[0] + p_ref[1]) * (1.0 / H)


def _merge(outp):
    B = 2000
    return pl.pallas_call(
        _merge_body,
        grid=(N // B,),
        in_specs=[pl.BlockSpec((NC, B, OUT), lambda i: (0, i, 0))],
        out_specs=pl.BlockSpec((B, OUT), lambda i: (i, 0)),
        out_shape=jax.ShapeDtypeStruct((N, OUT), jnp.float32),
    )(outp)


def _phase1(scores, epk1, z16):
    mesh = plsc.VectorSubcoreMesh(core_axis_name="c", subcore_axis_name="s", num_cores=NC, num_subcores=NS)
    NCH = EPW // C

    @functools.partial(
        pl.kernel,
        out_type=[
            jax.ShapeDtypeStruct((E, H), jnp.float32),
            jax.ShapeDtypeStruct((NC, N, 4 * H), jnp.float32),
        ],
        mesh=mesh,
        compiler_params=pltpu.CompilerParams(use_tc_tiling_on_sc=False, needs_layout_passes=False),
        scratch_types=[
            pltpu.VMEM((N, 2 * H), jnp.float32),
            pltpu.VMEM((2, 2, C), jnp.int32),
            pltpu.VMEM((C, H), jnp.float32),
            pltpu.VMEM((C, 4 * H), jnp.float32),
            pltpu.VMEM_SHARED((N, 4 * H), jnp.float32),
            pltpu.SemaphoreType.DMA,
            pltpu.SemaphoreType.DMA,
        ],
    )
    def k(scores_hbm, epk_hbm, z16_hbm, enum_hbm, sacc_hbm,
          tbl, eb, ev, evp, sacc_sh, se0, se1):
        cid = lax.axis_index("c")
        sid = lax.axis_index("s")
        wid = sid * NC + cid
        pltpu.sync_copy(scores_hbm, tbl)
        pltpu.sync_copy(z16_hbm.at[pl.ds(sid * NPW, NPW)],
                        sacc_sh.at[pl.ds(sid * NPW, NPW)])
        @pl.loop(0, C)
        def _z(i):
            evp[i, pl.ds(0, 16)] = jnp.zeros((16,), jnp.float32)

        plsc.subcore_barrier()

        base = wid * EPW
        lane = lax.iota(jnp.int32, 16)
        my_epk = epk_hbm.at[wid]
        sems_e = (se0, se1)

        def do_chunk(ck, cur, nxt, prefetch):
            if prefetch:
                pltpu.async_copy(my_epk.at[ck + 1], eb.at[nxt], sems_e[nxt])

            @pl.loop(0, C, step=16)
            def _grp(i):
                r16 = eb[cur, 0, pl.ds(i, 16)]
                c16 = eb[cur, 1, pl.ds(i, 16)]
                for hh in range(H):
                    hv = jnp.full((16,), hh, jnp.int32)
                    sl = plsc.load_gather(tbl, [r16, hv])
                    sr = plsc.load_gather(tbl, [c16, hv + H])
                    l = sl + sr
                    l = jnp.where(l >= 0.0, l, l * LEAK)
                    e = jnp.exp(l)
                    plsc.store_scatter(ev, [lane + i, hv], e)
                    plsc.store_scatter(evp, [lane + i, hv], e)

            pltpu.sync_copy(ev, enum_hbm.at[pl.ds(base + ck * C, C)])
            pltpu.sync_copy(evp, sacc_sh.at[eb.at[cur].at[0]], add=True)
            if prefetch:
                pltpu.make_async_copy(my_epk.at[0], eb.at[nxt],
                                      sems_e[nxt]).wait()

        pltpu.sync_copy(my_epk.at[0], eb.at[0])

        @pl.loop(0, NCH - 1, step=2)
        def _pair(g):
            do_chunk(g, 0, 1, True)
            do_chunk(g + 1, 1, 0, True)

        do_chunk(NCH - 1, 0, 1, False)

        plsc.subcore_barrier()
        pltpu.sync_copy(sacc_sh.at[pl.ds(sid * NPW, NPW)],
                        sacc_hbm.at[cid].at[pl.ds(sid * NPW, NPW)])

    return k(scores, epk1, z16)


def _phase2(hbf, rtab, enum, epk2, z128):
    mesh = plsc.VectorSubcoreMesh(core_axis_name="c", subcore_axis_name="s", num_cores=NC, num_subcores=NS)
    HC = C // 2
    NCH = EPW // C

    @functools.partial(
        pl.kernel,
        out_type=[
            jax.ShapeDtypeStruct((E, H), jnp.float32),
            jax.ShapeDtypeStruct((NC, N, OUT), jnp.float32),
        ],
        mesh=mesh,
        compiler_params=pltpu.CompilerParams(use_tc_tiling_on_sc=False, needs_layout_passes=False),
        scratch_types=[
            pltpu.VMEM((2, 2, C), jnp.int32),
            pltpu.VMEM((2, C, H), jnp.float32),
            pltpu.VMEM((2, C, H), jnp.float32),
            pltpu.VMEM((2, C, 4 * H), jnp.float32),
            pltpu.VMEM((HC, H * OUT), jnp.bfloat16),
            pltpu.VMEM((HC, H * OUT), jnp.bfloat16),
            pltpu.VMEM((C, OUT), jnp.float32),
            pltpu.VMEM_SHARED((N, OUT), jnp.float32),
            pltpu.SemaphoreType.DMA,
            pltpu.SemaphoreType.DMA,
            pltpu.SemaphoreType.DMA,
            pltpu.SemaphoreType.DMA,
            pltpu.SemaphoreType.DMA,
            pltpu.SemaphoreType.DMA,
            pltpu.SemaphoreType.DMA,
            pltpu.SemaphoreType.DMA,
        ],
    )
    def k(h_hbm, rtab_hbm, enum_hbm, epk_hbm, z128_hbm,
          alpha_hbm, outp_hbm,
          ebuf, evv, avv, rgath, rows0, rows1, contrib, acc_sh,
          se0, se1, sp0, sp1, sr0, sr1, sa0, sa1):
        cid = lax.axis_index("c")
        sid = lax.axis_index("s")
        wid = sid * NC + cid
        sems_e = (se0, se1)
        sems_p = (sp0, sp1)
        sems_r = (sr0, sr1)
        sems_a = (sa0, sa1)
        rowsb = (rows0, rows1)
        pltpu.sync_copy(z128_hbm.at[pl.ds(sid * NPW, NPW)],
                        acc_sh.at[pl.ds(sid * NPW, NPW)])
        plsc.subcore_barrier()

        base = wid * EPW
        lane = lax.iota(jnp.int32, 16)
        my_epk = epk_hbm.at[wid]

        def issue_pre(ck, s):
            pltpu.async_copy(enum_hbm.at[pl.ds(base + ck * C, C)],
                             evv.at[s], sems_p[s])
            pltpu.async_copy(rtab_hbm.at[ebuf.at[s].at[0]],
                             rgath.at[s], sems_p[s])

        def wait_pre(s):
            pltpu.make_async_copy(enum_hbm.at[pl.ds(0, C)],
                                  evv.at[s], sems_p[s]).wait()
            pltpu.make_async_copy(rtab_hbm.at[ebuf.at[s].at[0]],
                                  rgath.at[s], sems_p[s]).wait()

        def issue_rows(s, hf):
            pltpu.async_copy(
                h_hbm.at[ebuf.at[s].at[1].at[pl.ds(hf * HC, HC)]],
                rowsb[hf], sems_r[hf])

        def wait_rows(s, hf):
            pltpu.make_async_copy(
                h_hbm.at[ebuf.at[s].at[1].at[pl.ds(hf * HC, HC)]],
                rowsb[hf], sems_r[hf]).wait()

        def issue_ebuf(ck, s):
            pltpu.async_copy(my_epk.at[ck], ebuf.at[s], sems_e[s])

        def wait_ebuf(s):
            pltpu.make_async_copy(my_epk.at[0], ebuf.at[s], sems_e[s]).wait()

        def issue_aout(ck, s):
            pltpu.async_copy(avv.at[s], alpha_hbm.at[pl.ds(base + ck * C, C)],
                             sems_a[s])

        def wait_aout(s):
            pltpu.make_async_copy(avv.at[s], alpha_hbm.at[pl.ds(0, C)],
                                  sems_a[s]).wait()

        def do_chunk(ck, cur, nxt, prefetch, prefetch2):
            if prefetch:
                wait_ebuf(nxt)
                issue_pre(ck + 1, nxt)
            wait_pre(cur)
            wait_aout(cur)

            @pl.loop(0, C, step=16)
            def _grp(i):
                for hh in range(H):
                    hv = jnp.full((16,), hh, jnp.int32)
                    rv = plsc.load_gather(rgath.at[cur], [lane + i, hv])
                    en = plsc.load_gather(evv.at[cur], [lane + i, hv])
                    plsc.store_scatter(avv.at[cur], [lane + i, hv], en * rv)

            issue_aout(ck, cur)

            for hf in range(2):
                wait_rows(cur, hf)

                @pl.loop(0, HC)
                def _edge(e, _h=hf):
                    ge = _h * HC + e
                    ev16 = jnp.full((16,), ge, jnp.int32)
                    ab = [plsc.load_gather(
                              avv.at[cur],
                              [ev16, jnp.full((16,), hh, jnp.int32)])
                          for hh in range(H)]
                    rr = rowsb[_h]
                    for ob in range(OUT // 32):
                        acc_a = acc_b = None
                        for hh in range(H):
                            g = hh * (OUT // 32) + ob
                            a, b = plsc.unpack(
                                rr[e, pl.ds(g * 32, 32)],
                                format=plsc.PackFormat.INTERLEAVED)
                            if hh == 0:
                                acc_a = ab[0] * a
                                acc_b = ab[0] * b
                            else:
                                acc_a = acc_a + ab[hh] * a
                                acc_b = acc_b + ab[hh] * b
                        contrib[ge, pl.ds(ob * 32, 16)] = acc_a
                        contrib[ge, pl.ds(ob * 32 + 16, 16)] = acc_b

                if prefetch:
                    issue_rows(nxt, hf)

            pltpu.sync_copy(contrib, acc_sh.at[ebuf.at[cur].at[0]], add=True)
            if prefetch2 is not None:
                issue_ebuf(prefetch2, cur)

        pltpu.sync_copy(my_epk.at[0], ebuf.at[0])
        issue_pre(0, 0)
        issue_rows(0, 0)
        issue_rows(0, 1)
        issue_ebuf(1, 1)
        issue_aout(0, 0)
        issue_aout(1, 1)

        @pl.loop(0, NCH - 1, step=2)
        def _pair(g):
            do_chunk(g, 0, 1, True, g + 2)
            do_chunk(g + 1, 1, 0, True, jnp.minimum(g + 3, NCH - 1))

        do_chunk(NCH - 1, 0, 1, False, None)
        wait_ebuf(1)
        wait_aout(0)
        wait_aout(1)

        plsc.subcore_barrier()
        pltpu.sync_copy(acc_sh.at[pl.ds(sid * NPW, NPW)],
                        outp_hbm.at[cid].at[pl.ds(sid * NPW, NPW)])

    return k(hbf, rtab, enum, epk2, z128)


def kernel(x, edge_index, W, attn):
    row = edge_index[0]
    col = edge_index[1]
    wt = W.T
    attn_l = attn[:, :OUT]
    attn_r = attn[:, OUT:]
    w3 = wt.reshape(IN, H, OUT)
    wl = jnp.einsum("khj,hj->kh", w3, attn_l)
    wr = jnp.einsum("khj,hj->kh", w3, attn_r)
    wlr = jnp.concatenate([wl, wr], axis=1)

    NCH = EPW // C
    epk = jnp.concatenate([row.reshape(NW, NCH, 1, C),
                           col.reshape(NW, NCH, 1, C)], axis=2)

    h, scores = _proj(x, wt, wlr)
    hbf = h.reshape(N, 16, 2, 16).swapaxes(2, 3).reshape(N, H * OUT)
    hbf = hbf.astype(jnp.bfloat16)
    z16 = jnp.zeros((N, 4 * H), jnp.float32)
    z128 = jnp.zeros((N, OUT), jnp.float32)

    enum, sacc = _phase1(scores, epk, z16)
    rtab = _recip(sacc)
    alpha, outp = _phase2(hbf, rtab, enum, epk, z128)
    out = _merge(outp)
    return out, alpha


# speedup vs baseline: 38.3292x; 1.0078x over previous
import functools

import jax
import jax.numpy as jnp
from jax import lax
from jax.experimental import pallas as pl
from jax.experimental.pallas import tpu as pltpu
from jax.experimental.pallas import tpu_sc as plsc

N = 10000
E = 320000
IN = 128
OUT = 128
H = 4

NC = 2
NS = 16
NW = NC * NS
EPW = E // NW
NPW = N // NS
C = 80
LEAK = 0.2


def _proj_body(x_ref, wt_ref, wlr_ref, h_ref, s_ref):
    xb = x_ref[...]
    h_ref[...] = lax.dot_general(
        xb, wt_ref[...], (((1,), (0,)), ((), ())),
        preferred_element_type=jnp.float32)
    s_ref[...] = lax.dot_general(
        xb, wlr_ref[...], (((1,), (0,)), ((), ())),
        preferred_element_type=jnp.float32)


def _proj(x, wt, wlr):
    B = 2000
    return pl.pallas_call(
        _proj_body,
        grid=(N // B,),
        in_specs=[
            pl.BlockSpec((B, IN), lambda i: (i, 0)),
            pl.BlockSpec((IN, H * OUT), lambda i: (0, 0)),
            pl.BlockSpec((IN, 2 * H), lambda i: (0, 0)),
        ],
        out_specs=[
            pl.BlockSpec((B, H * OUT), lambda i: (i, 0)),
            pl.BlockSpec((B, 2 * H), lambda i: (i, 0)),
        ],
        out_shape=[
            jax.ShapeDtypeStruct((N, H * OUT), jnp.float32),
            jax.ShapeDtypeStruct((N, 2 * H), jnp.float32),
        ],
    )(x, wt, wlr)


def _merge_body(p_ref, o_ref):
    o_ref[...] = (p_ref[0] + p_ref[1]) * (1.0 / H)


def _merge(outp):
    B = 2000
    return pl.pallas_call(
        _merge_body,
        grid=(N // B,),
        in_specs=[pl.BlockSpec((NC, B, OUT), lambda i: (0, i, 0))],
        out_specs=pl.BlockSpec((B, OUT), lambda i: (i, 0)),
        out_shape=jax.ShapeDtypeStruct((N, OUT), jnp.float32),
    )(outp)


def _phase1(scores, epk1, z16):
    mesh = plsc.VectorSubcoreMesh(core_axis_name="c", subcore_axis_name="s", num_cores=NC, num_subcores=NS)
    NCH = EPW // C

    @functools.partial(
        pl.kernel,
        out_type=[
            jax.ShapeDtypeStruct((E, H), jnp.float32),
            jax.ShapeDtypeStruct((NC, N, 4 * H), jnp.float32),
        ],
        mesh=mesh,
        compiler_params=pltpu.CompilerParams(use_tc_tiling_on_sc=False, needs_layout_passes=False),
        scratch_types=[
            pltpu.VMEM((N, 2 * H), jnp.float32),
            pltpu.VMEM((2, 2, C), jnp.int32),
            pltpu.VMEM((C, H), jnp.float32),
            pltpu.VMEM((C, 4 * H), jnp.float32),
            pltpu.VMEM_SHARED((N, 4 * H), jnp.float32),
            pltpu.SemaphoreType.DMA,
            pltpu.SemaphoreType.DMA,
        ],
    )
    def k(scores_hbm, epk_hbm, z16_hbm, enum_hbm, sacc_hbm,
          tbl, eb, ev, evp, sacc_sh, se0, se1):
        cid = lax.axis_index("c")
        sid = lax.axis_index("s")
        wid = sid * NC + cid
        pltpu.sync_copy(scores_hbm, tbl)
        pltpu.sync_copy(z16_hbm.at[pl.ds(sid * NPW, NPW)],
                        sacc_sh.at[pl.ds(sid * NPW, NPW)])
        @pl.loop(0, C)
        def _z(i):
            evp[i, pl.ds(0, 16)] = jnp.zeros((16,), jnp.float32)

        plsc.subcore_barrier()

        base = wid * EPW
        lane = lax.iota(jnp.int32, 16)
        my_epk = epk_hbm.at[wid]
        sems_e = (se0, se1)

        def do_chunk(ck, cur, nxt, prefetch):
            if prefetch:
                pltpu.async_copy(my_epk.at[ck + 1], eb.at[nxt], sems_e[nxt])

            @pl.loop(0, C, step=16)
            def _grp(i):
                r16 = eb[cur, 0, pl.ds(i, 16)]
                c16 = eb[cur, 1, pl.ds(i, 16)]
                for hh in range(H):
                    hv = jnp.full((16,), hh, jnp.int32)
                    sl = plsc.load_gather(tbl, [r16, hv])
                    sr = plsc.load_gather(tbl, [c16, hv + H])
                    l = sl + sr
                    l = jnp.where(l >= 0.0, l, l * LEAK)
                    e = jnp.exp(l)
                    plsc.store_scatter(ev, [lane + i, hv], e)
                    plsc.store_scatter(evp, [lane + i, hv], e)

            pltpu.sync_copy(ev, enum_hbm.at[pl.ds(base + ck * C, C)])
            pltpu.sync_copy(evp, sacc_sh.at[eb.at[cur].at[0]], add=True)
            if prefetch:
                pltpu.make_async_copy(my_epk.at[0], eb.at[nxt],
                                      sems_e[nxt]).wait()

        pltpu.sync_copy(my_epk.at[0], eb.at[0])

        @pl.loop(0, NCH - 1, step=2)
        def _pair(g):
            do_chunk(g, 0, 1, True)
            do_chunk(g + 1, 1, 0, True)

        do_chunk(NCH - 1, 0, 1, False)

        plsc.subcore_barrier()
        pltpu.sync_copy(sacc_sh.at[pl.ds(sid * NPW, NPW)],
                        sacc_hbm.at[cid].at[pl.ds(sid * NPW, NPW)])

    return k(scores, epk1, z16)


def _phase2(hbf, sacc, enum, epk2, z128):
    mesh = plsc.VectorSubcoreMesh(core_axis_name="c", subcore_axis_name="s", num_cores=NC, num_subcores=NS)
    HC = C // 2
    NCH = EPW // C

    @functools.partial(
        pl.kernel,
        out_type=[
            jax.ShapeDtypeStruct((E, H), jnp.float32),
            jax.ShapeDtypeStruct((NC, N, OUT), jnp.float32),
        ],
        mesh=mesh,
        compiler_params=pltpu.CompilerParams(use_tc_tiling_on_sc=False, needs_layout_passes=False),
        scratch_types=[
            pltpu.VMEM((2, 2, C), jnp.int32),
            pltpu.VMEM((2, C), jnp.int32),
            pltpu.VMEM((2, C, H), jnp.float32),
            pltpu.VMEM((2, C, H), jnp.float32),
            pltpu.VMEM((2, C, 4 * H), jnp.float32),
            pltpu.VMEM((2, C, 4 * H), jnp.float32),
            pltpu.VMEM((HC, H * OUT), jnp.bfloat16),
            pltpu.VMEM((HC, H * OUT), jnp.bfloat16),
            pltpu.VMEM((2, C, OUT), jnp.float32),
            pltpu.VMEM_SHARED((N, OUT), jnp.float32),
            pltpu.SemaphoreType.DMA,
            pltpu.SemaphoreType.DMA,
            pltpu.SemaphoreType.DMA,
            pltpu.SemaphoreType.DMA,
            pltpu.SemaphoreType.DMA,
            pltpu.SemaphoreType.DMA,
            pltpu.SemaphoreType.DMA,
            pltpu.SemaphoreType.DMA,
            pltpu.SemaphoreType.DMA,
            pltpu.SemaphoreType.DMA,
            pltpu.SemaphoreType.DMA,
            pltpu.SemaphoreType.DMA,
        ],
    )
    def k(h_hbm, sacc_hbm, enum_hbm, epk_hbm, z128_hbm,
          alpha_hbm, outp_hbm,
          ebuf, sidx, evv, avv, rg0, rg1, rows0, rows1, contrib, acc_sh,
          se0, se1, sp0, sp1, sr0, sr1, sa0, sa1, si0, si1, sc0, sc1):
        cid = lax.axis_index("c")
        sid = lax.axis_index("s")
        wid = sid * NC + cid
        sems_e = (se0, se1)
        sems_p = (sp0, sp1)
        sems_r = (sr0, sr1)
        sems_a = (sa0, sa1)
        sems_i = (si0, si1)
        sems_s = (sc0, sc1)
        rowsb = (rows0, rows1)
        pltpu.sync_copy(z128_hbm.at[pl.ds(sid * NPW, NPW)],
                        acc_sh.at[pl.ds(sid * NPW, NPW)])
        plsc.subcore_barrier()

        base = wid * EPW
        lane = lax.iota(jnp.int32, 16)
        my_epk = epk_hbm.at[wid]

        def issue_pre(ck, s):
            pltpu.async_copy(enum_hbm.at[pl.ds(base + ck * C, C)],
                             evv.at[s], sems_p[s])
            pltpu.async_copy(sacc_hbm.at[0].at[ebuf.at[s].at[0]],
                             rg0.at[s], sems_p[s])
            pltpu.async_copy(sacc_hbm.at[1].at[ebuf.at[s].at[0]],
                             rg1.at[s], sems_p[s])

        def wait_pre(s):
            pltpu.make_async_copy(enum_hbm.at[pl.ds(0, C)],
                                  evv.at[s], sems_p[s]).wait()
            pltpu.make_async_copy(sacc_hbm.at[0].at[ebuf.at[s].at[0]],
                                  rg0.at[s], sems_p[s]).wait()
            pltpu.make_async_copy(sacc_hbm.at[1].at[ebuf.at[s].at[0]],
                                  rg1.at[s], sems_p[s]).wait()

        def issue_rows(s, hf):
            pltpu.async_copy(
                h_hbm.at[ebuf.at[s].at[1].at[pl.ds(hf * HC, HC)]],
                rowsb[hf], sems_r[hf])

        def wait_rows(s, hf):
            pltpu.make_async_copy(
                h_hbm.at[ebuf.at[s].at[1].at[pl.ds(hf * HC, HC)]],
                rowsb[hf], sems_r[hf]).wait()

        def issue_ebuf(ck, s):
            pltpu.async_copy(my_epk.at[ck], ebuf.at[s], sems_e[s])

        def wait_ebuf(s):
            pltpu.make_async_copy(my_epk.at[0], ebuf.at[s], sems_e[s]).wait()

        def issue_aout(ck, s):
            pltpu.async_copy(avv.at[s], alpha_hbm.at[pl.ds(base + ck * C, C)],
                             sems_a[s])

        def wait_aout(s):
            pltpu.make_async_copy(avv.at[s], alpha_hbm.at[pl.ds(0, C)],
                                  sems_a[s]).wait()

        def issue_sidx(ck, s):
            pltpu.async_copy(my_epk.at[ck].at[0], sidx.at[s], sems_i[s])

        def wait_sidx(s):
            pltpu.make_async_copy(my_epk.at[0].at[0], sidx.at[s],
                                  sems_i[s]).wait()

        def issue_scatter(s):
            pltpu.async_copy(contrib.at[s], acc_sh.at[sidx.at[s]],
                             sems_s[s], add=True)

        def wait_scatter(s):
            pltpu.make_async_copy(contrib.at[s], acc_sh.at[sidx.at[s]],
                                  sems_s[s]).wait()

        def do_chunk(ck, cur, nxt, prefetch, prefetch2):
            if prefetch:
                wait_ebuf(nxt)
                issue_pre(ck + 1, nxt)
            wait_pre(cur)
            wait_aout(cur)

            @pl.loop(0, C, step=16)
            def _grp(i):
                for hh in range(H):
                    hv = jnp.full((16,), hh, jnp.int32)
                    s0 = plsc.load_gather(rg0.at[cur], [lane + i, hv])
                    s1 = plsc.load_gather(rg1.at[cur], [lane + i, hv])
                    en = plsc.load_gather(evv.at[cur], [lane + i, hv])
                    plsc.store_scatter(avv.at[cur], [lane + i, hv],
                                       en / (s0 + s1 + 1e-16))

            issue_aout(ck, cur)

            @pl.when(ck > 0)
            def _ws():
                wait_scatter(nxt)

            if prefetch:
                issue_sidx(ck + 1, nxt)

            for hf in range(2):
                wait_rows(cur, hf)

                @pl.loop(0, HC)
                def _edge(e, _h=hf):
                    ge = _h * HC + e
                    ev16 = jnp.full((16,), ge, jnp.int32)
                    ab = [plsc.load_gather(
                              avv.at[cur],
                              [ev16, jnp.full((16,), hh, jnp.int32)])
                          for hh in range(H)]
                    rr = rowsb[_h]
                    cc = contrib.at[cur]
                    for ob in range(OUT // 32):
                        acc_a = acc_b = None
                        for hh in range(H):
                            g = hh * (OUT // 32) + ob
                            a, b = plsc.unpack(
                                rr[e, pl.ds(g * 32, 32)],
                                format=plsc.PackFormat.INTERLEAVED)
                            if hh == 0:
                                acc_a = ab[0] * a
                                acc_b = ab[0] * b
                            else:
                                acc_a = acc_a + ab[hh] * a
                                acc_b = acc_b + ab[hh] * b
                        cc[ge, pl.ds(ob * 32, 16)] = acc_a
                        cc[ge, pl.ds(ob * 32 + 16, 16)] = acc_b

                if prefetch:
                    issue_rows(nxt, hf)

            wait_sidx(cur)
            issue_scatter(cur)
            if prefetch2 is not None:
                issue_ebuf(prefetch2, cur)

        pltpu.sync_copy(my_epk.at[0], ebuf.at[0])
        issue_pre(0, 0)
        issue_rows(0, 0)
        issue_rows(0, 1)
        issue_ebuf(1, 1)
        issue_sidx(0, 0)
        issue_aout(0, 0)
        issue_aout(1, 1)

        @pl.loop(0, NCH - 1, step=2)
        def _pair(g):
            do_chunk(g, 0, 1, True, g + 2)
            do_chunk(g + 1, 1, 0, True, jnp.minimum(g + 3, NCH - 1))

        do_chunk(NCH - 1, 0, 1, False, None)
        wait_ebuf(1)
        wait_aout(0)
        wait_aout(1)
        wait_scatter(0)

        plsc.subcore_barrier()
        pltpu.sync_copy(acc_sh.at[pl.ds(sid * NPW, NPW)],
                        outp_hbm.at[cid].at[pl.ds(sid * NPW, NPW)])

    return k(hbf, sacc, enum, epk2, z128)


def kernel(x, edge_index, W, attn):
    row = edge_index[0]
    col = edge_index[1]
    wt = W.T
    attn_l = attn[:, :OUT]
    attn_r = attn[:, OUT:]
    w3 = wt.reshape(IN, H, OUT)
    wl = jnp.einsum("khj,hj->kh", w3, attn_l)
    wr = jnp.einsum("khj,hj->kh", w3, attn_r)
    wlr = jnp.concatenate([wl, wr], axis=1)

    NCH = EPW // C
    epk = jnp.concatenate([row.reshape(NW, NCH, 1, C),
                           col.reshape(NW, NCH, 1, C)], axis=2)

    h, scores = _proj(x, wt, wlr)
    hbf = h.reshape(N, 16, 2, 16).swapaxes(2, 3).reshape(N, H * OUT)
    hbf = hbf.astype(jnp.bfloat16)
    z16 = jnp.zeros((N, 4 * H), jnp.float32)
    z128 = jnp.zeros((N, OUT), jnp.float32)

    enum, sacc = _phase1(scores, epk, z16)
    alpha, outp = _phase2(hbf, sacc, enum, epk, z128)
    out = _merge(outp)
    return out, alpha


# speedup vs baseline: 43.8026x; 1.1428x over previous
import functools

import jax
import jax.numpy as jnp
from jax import lax
from jax.experimental import pallas as pl
from jax.experimental.pallas import tpu as pltpu
from jax.experimental.pallas import tpu_sc as plsc

N = 10000
E = 320000
IN = 128
OUT = 128
H = 4

NC = 2
NS = 16
NW = NC * NS
EPW = E // NW
NPW = N // NS
C = 80
LEAK = 0.2


def _proj_body(x_ref, wtp_ref, wlr_ref, hb_ref, s_ref):
    xb = x_ref[...]
    hp = lax.dot_general(
        xb, wtp_ref[...], (((1,), (0,)), ((), ())),
        preferred_element_type=jnp.float32)
    hb_ref[...] = hp.astype(jnp.bfloat16)
    s_ref[...] = lax.dot_general(
        xb, wlr_ref[...], (((1,), (0,)), ((), ())),
        preferred_element_type=jnp.float32)


def _proj(x, wtp, wlr):
    B = 2000
    return pl.pallas_call(
        _proj_body,
        grid=(N // B,),
        in_specs=[
            pl.BlockSpec((B, IN), lambda i: (i, 0)),
            pl.BlockSpec((IN, H * OUT), lambda i: (0, 0)),
            pl.BlockSpec((IN, 2 * H), lambda i: (0, 0)),
        ],
        out_specs=[
            pl.BlockSpec((B, H * OUT), lambda i: (i, 0)),
            pl.BlockSpec((B, 2 * H), lambda i: (i, 0)),
        ],
        out_shape=[
            jax.ShapeDtypeStruct((N, H * OUT), jnp.bfloat16),
            jax.ShapeDtypeStruct((N, 2 * H), jnp.float32),
        ],
    )(x, wtp, wlr)


def _merge_body(p_ref, o_ref):
    o_ref[...] = (p_ref[0] + p_ref[1]) * (1.0 / H)


def _merge(outp):
    B = 2000
    return pl.pallas_call(
        _merge_body,
        grid=(N // B,),
        in_specs=[pl.BlockSpec((NC, B, OUT), lambda i: (0, i, 0))],
        out_specs=pl.BlockSpec((B, OUT), lambda i: (i, 0)),
        out_shape=jax.ShapeDtypeStruct((N, OUT), jnp.float32),
    )(outp)


def _phase1(scores, epk1, z16):
    mesh = plsc.VectorSubcoreMesh(core_axis_name="c", subcore_axis_name="s", num_cores=NC, num_subcores=NS)
    NCH = EPW // C

    @functools.partial(
        pl.kernel,
        out_type=[
            jax.ShapeDtypeStruct((E, H), jnp.float32),
            jax.ShapeDtypeStruct((NC, N, 4 * H), jnp.float32),
        ],
        mesh=mesh,
        compiler_params=pltpu.CompilerParams(use_tc_tiling_on_sc=False, needs_layout_passes=False),
        scratch_types=[
            pltpu.VMEM((N, 2 * H), jnp.float32),
            pltpu.VMEM((2, 2, C), jnp.int32),
            pltpu.VMEM((C, H), jnp.float32),
            pltpu.VMEM((C, 4 * H), jnp.float32),
            pltpu.VMEM_SHARED((N, 4 * H), jnp.float32),
            pltpu.SemaphoreType.DMA,
            pltpu.SemaphoreType.DMA,
        ],
    )
    def k(scores_hbm, epk_hbm, z16_hbm, enum_hbm, sacc_hbm,
          tbl, eb, ev, evp, sacc_sh, se0, se1):
        cid = lax.axis_index("c")
        sid = lax.axis_index("s")
        wid = sid * NC + cid
        pltpu.sync_copy(scores_hbm, tbl)
        pltpu.sync_copy(z16_hbm.at[pl.ds(sid * NPW, NPW)],
                        sacc_sh.at[pl.ds(sid * NPW, NPW)])
        @pl.loop(0, C)
        def _z(i):
            evp[i, pl.ds(0, 16)] = jnp.zeros((16,), jnp.float32)

        plsc.subcore_barrier()

        base = wid * EPW
        lane = lax.iota(jnp.int32, 16)
        my_epr = epk_hbm.at[0].at[wid]
        my_epc = epk_hbm.at[1].at[wid]
        sems_e = (se0, se1)

        def do_chunk(ck, cur, nxt, prefetch):
            if prefetch:
                pltpu.async_copy(my_epr.at[ck + 1], eb.at[nxt].at[0], sems_e[nxt])
                pltpu.async_copy(my_epc.at[ck + 1], eb.at[nxt].at[1], sems_e[nxt])

            @pl.loop(0, C, step=16)
            def _grp(i):
                r16 = eb[cur, 0, pl.ds(i, 16)]
                c16 = eb[cur, 1, pl.ds(i, 16)]
                for hh in range(H):
                    hv = jnp.full((16,), hh, jnp.int32)
                    sl = plsc.load_gather(tbl, [r16, hv])
                    sr = plsc.load_gather(tbl, [c16, hv + H])
                    l = sl + sr
                    l = jnp.where(l >= 0.0, l, l * LEAK)
                    e = jnp.exp(l)
                    plsc.store_scatter(ev, [lane + i, hv], e)
                    plsc.store_scatter(evp, [lane + i, hv], e)

            pltpu.sync_copy(ev, enum_hbm.at[pl.ds(base + ck * C, C)])
            pltpu.sync_copy(evp, sacc_sh.at[eb.at[cur].at[0]], add=True)
            if prefetch:
                pltpu.make_async_copy(my_epr.at[0], eb.at[nxt].at[0],
                                      sems_e[nxt]).wait()
                pltpu.make_async_copy(my_epc.at[0], eb.at[nxt].at[1],
                                      sems_e[nxt]).wait()

        pltpu.sync_copy(my_epr.at[0], eb.at[0].at[0])
        pltpu.sync_copy(my_epc.at[0], eb.at[0].at[1])

        @pl.loop(0, NCH - 1, step=2)
        def _pair(g):
            do_chunk(g, 0, 1, True)
            do_chunk(g + 1, 1, 0, True)

        do_chunk(NCH - 1, 0, 1, False)

        plsc.subcore_barrier()
        pltpu.sync_copy(sacc_sh.at[pl.ds(sid * NPW, NPW)],
                        sacc_hbm.at[cid].at[pl.ds(sid * NPW, NPW)])

    return k(scores, epk1, z16)


def _phase2(hbf, sacc, enum, epk2, z128):
    mesh = plsc.VectorSubcoreMesh(core_axis_name="c", subcore_axis_name="s", num_cores=NC, num_subcores=NS)
    HC = C // 2
    NCH = EPW // C

    @functools.partial(
        pl.kernel,
        out_type=[
            jax.ShapeDtypeStruct((E, H), jnp.float32),
            jax.ShapeDtypeStruct((NC, N, OUT), jnp.float32),
        ],
        mesh=mesh,
        compiler_params=pltpu.CompilerParams(use_tc_tiling_on_sc=False, needs_layout_passes=False),
        scratch_types=[
            pltpu.VMEM((2, 2, C), jnp.int32),
            pltpu.VMEM((2, C), jnp.int32),
            pltpu.VMEM((2, C, H), jnp.float32),
            pltpu.VMEM((2, C, H), jnp.float32),
            pltpu.VMEM((2, C, 4 * H), jnp.float32),
            pltpu.VMEM((2, C, 4 * H), jnp.float32),
            pltpu.VMEM((HC, H * OUT), jnp.bfloat16),
            pltpu.VMEM((HC, H * OUT), jnp.bfloat16),
            pltpu.VMEM((2, C, OUT), jnp.float32),
            pltpu.VMEM_SHARED((N, OUT), jnp.float32),
            pltpu.SemaphoreType.DMA,
            pltpu.SemaphoreType.DMA,
            pltpu.SemaphoreType.DMA,
            pltpu.SemaphoreType.DMA,
            pltpu.SemaphoreType.DMA,
            pltpu.SemaphoreType.DMA,
            pltpu.SemaphoreType.DMA,
            pltpu.SemaphoreType.DMA,
            pltpu.SemaphoreType.DMA,
            pltpu.SemaphoreType.DMA,
            pltpu.SemaphoreType.DMA,
            pltpu.SemaphoreType.DMA,
        ],
    )
    def k(h_hbm, sacc_hbm, enum_hbm, epk_hbm, z128_hbm,
          alpha_hbm, outp_hbm,
          ebuf, sidx, evv, avv, rg0, rg1, rows0, rows1, contrib, acc_sh,
          se0, se1, sp0, sp1, sr0, sr1, sa0, sa1, si0, si1, sc0, sc1):
        cid = lax.axis_index("c")
        sid = lax.axis_index("s")
        wid = sid * NC + cid
        sems_e = (se0, se1)
        sems_p = (sp0, sp1)
        sems_r = (sr0, sr1)
        sems_a = (sa0, sa1)
        sems_i = (si0, si1)
        sems_s = (sc0, sc1)
        rowsb = (rows0, rows1)
        pltpu.sync_copy(z128_hbm.at[pl.ds(sid * NPW, NPW)],
                        acc_sh.at[pl.ds(sid * NPW, NPW)])
        plsc.subcore_barrier()

        base = wid * EPW
        lane = lax.iota(jnp.int32, 16)
        my_epr = epk_hbm.at[0].at[wid]
        my_epc = epk_hbm.at[1].at[wid]

        def issue_pre(ck, s):
            pltpu.async_copy(enum_hbm.at[pl.ds(base + ck * C, C)],
                             evv.at[s], sems_p[s])
            pltpu.async_copy(sacc_hbm.at[0].at[ebuf.at[s].at[0]],
                             rg0.at[s], sems_p[s])
            pltpu.async_copy(sacc_hbm.at[1].at[ebuf.at[s].at[0]],
                             rg1.at[s], sems_p[s])

        def wait_pre(s):
            pltpu.make_async_copy(enum_hbm.at[pl.ds(0, C)],
                                  evv.at[s], sems_p[s]).wait()
            pltpu.make_async_copy(sacc_hbm.at[0].at[ebuf.at[s].at[0]],
                                  rg0.at[s], sems_p[s]).wait()
            pltpu.make_async_copy(sacc_hbm.at[1].at[ebuf.at[s].at[0]],
                                  rg1.at[s], sems_p[s]).wait()

        def issue_rows(s, hf):
            pltpu.async_copy(
                h_hbm.at[ebuf.at[s].at[1].at[pl.ds(hf * HC, HC)]],
                rowsb[hf], sems_r[hf])

        def wait_rows(s, hf):
            pltpu.make_async_copy(
                h_hbm.at[ebuf.at[s].at[1].at[pl.ds(hf * HC, HC)]],
                rowsb[hf], sems_r[hf]).wait()

        def issue_ebuf(ck, s):
            pltpu.async_copy(my_epr.at[ck], ebuf.at[s].at[0], sems_e[s])
            pltpu.async_copy(my_epc.at[ck], ebuf.at[s].at[1], sems_e[s])

        def wait_ebuf(s):
            pltpu.make_async_copy(my_epr.at[0], ebuf.at[s].at[0],
                                  sems_e[s]).wait()
            pltpu.make_async_copy(my_epc.at[0], ebuf.at[s].at[1],
                                  sems_e[s]).wait()

        def issue_aout(ck, s):
            pltpu.async_copy(avv.at[s], alpha_hbm.at[pl.ds(base + ck * C, C)],
                             sems_a[s])

        def wait_aout(s):
            pltpu.make_async_copy(avv.at[s], alpha_hbm.at[pl.ds(0, C)],
                                  sems_a[s]).wait()

        def issue_sidx(ck, s):
            pltpu.async_copy(my_epr.at[ck], sidx.at[s], sems_i[s])

        def wait_sidx(s):
            pltpu.make_async_copy(my_epr.at[0], sidx.at[s],
                                  sems_i[s]).wait()

        def issue_scatter(s):
            pltpu.async_copy(contrib.at[s], acc_sh.at[sidx.at[s]],
                             sems_s[s], add=True)

        def wait_scatter(s):
            pltpu.make_async_copy(contrib.at[s], acc_sh.at[sidx.at[s]],
                                  sems_s[s]).wait()

        def do_chunk(ck, cur, nxt, prefetch, prefetch2):
            if prefetch:
                wait_ebuf(nxt)
                issue_pre(ck + 1, nxt)
            wait_pre(cur)
            wait_aout(cur)

            @pl.loop(0, C, step=16)
            def _grp(i):
                for hh in range(H):
                    hv = jnp.full((16,), hh, jnp.int32)
                    s0 = plsc.load_gather(rg0.at[cur], [lane + i, hv])
                    s1 = plsc.load_gather(rg1.at[cur], [lane + i, hv])
                    en = plsc.load_gather(evv.at[cur], [lane + i, hv])
                    plsc.store_scatter(avv.at[cur], [lane + i, hv],
                                       en / (s0 + s1 + 1e-16))

            issue_aout(ck, cur)

            @pl.when(ck > 0)
            def _ws():
                wait_scatter(nxt)

            if prefetch:
                issue_sidx(ck + 1, nxt)

            for hf in range(2):
                wait_rows(cur, hf)

                @pl.loop(0, HC)
                def _edge(e, _h=hf):
                    ge = _h * HC + e
                    ev16 = jnp.full((16,), ge, jnp.int32)
                    ab = [plsc.load_gather(
                              avv.at[cur],
                              [ev16, jnp.full((16,), hh, jnp.int32)])
                          for hh in range(H)]
                    rr = rowsb[_h]
                    cc = contrib.at[cur]
                    for ob in range(OUT // 32):
                        acc_a = acc_b = None
                        for hh in range(H):
                            g = hh * (OUT // 32) + ob
                            a, b = plsc.unpack(
                                rr[e, pl.ds(g * 32, 32)],
                                format=plsc.PackFormat.INTERLEAVED)
                            if hh == 0:
                                acc_a = ab[0] * a
                                acc_b = ab[0] * b
                            else:
                                acc_a = acc_a + ab[hh] * a
                                acc_b = acc_b + ab[hh] * b
                        cc[ge, pl.ds(ob * 32, 16)] = acc_a
                        cc[ge, pl.ds(ob * 32 + 16, 16)] = acc_b

                if prefetch:
                    issue_rows(nxt, hf)

            wait_sidx(cur)
            issue_scatter(cur)
            if prefetch2 is not None:
                issue_ebuf(prefetch2, cur)

        pltpu.sync_copy(my_epr.at[0], ebuf.at[0].at[0])
        pltpu.sync_copy(my_epc.at[0], ebuf.at[0].at[1])
        issue_pre(0, 0)
        issue_rows(0, 0)
        issue_rows(0, 1)
        issue_ebuf(1, 1)
        issue_sidx(0, 0)
        issue_aout(0, 0)
        issue_aout(1, 1)

        @pl.loop(0, NCH - 1, step=2)
        def _pair(g):
            do_chunk(g, 0, 1, True, g + 2)
            do_chunk(g + 1, 1, 0, True, jnp.minimum(g + 3, NCH - 1))

        do_chunk(NCH - 1, 0, 1, False, None)
        wait_ebuf(1)
        wait_aout(0)
        wait_aout(1)
        wait_scatter(0)

        plsc.subcore_barrier()
        pltpu.sync_copy(acc_sh.at[pl.ds(sid * NPW, NPW)],
                        outp_hbm.at[cid].at[pl.ds(sid * NPW, NPW)])

    return k(hbf, sacc, enum, epk2, z128)


def kernel(x, edge_index, W, attn):
    row = edge_index[0]
    col = edge_index[1]
    wt = W.T
    attn_l = attn[:, :OUT]
    attn_r = attn[:, OUT:]
    w3 = wt.reshape(IN, H, OUT)
    wl = jnp.einsum("khj,hj->kh", w3, attn_l)
    wr = jnp.einsum("khj,hj->kh", w3, attn_r)
    wlr = jnp.concatenate([wl, wr], axis=1)

    NCH = EPW // C
    epk = edge_index.reshape(2, NW, NCH, C)

    ar = jnp.arange(H * OUT)
    perm = (ar // 32) * 32 + 16 * (ar % 2) + (ar % 32) // 2
    wtp = wt[:, perm]

    hbf, scores = _proj(x, wtp, wlr)
    z16 = jnp.zeros((N, 4 * H), jnp.float32)
    z128 = jnp.zeros((N, OUT), jnp.float32)

    enum, sacc = _phase1(scores, epk, z16)
    alpha, outp = _phase2(hbf, sacc, enum, epk, z128)
    out = _merge(outp)
    return out, alpha
